# Initial kernel scaffold; baseline (speedup 1.0000x reference)
import jax
import jax.numpy as jnp
from jax.experimental import pallas as pl


def kernel(q1, k1, q2, k2, temp):
    raise NotImplementedError("write your pallas kernel here")


# speedup vs baseline: 2.7699x; 2.7699x over previous
import functools

import jax
import jax.numpy as jnp
from jax import lax
from jax.experimental import pallas as pl
from jax.experimental.pallas import tpu as pltpu

B, N, D = 64, 32768, 64
TOPK = 25
L = 4096
NT = N // L

_PREC = jax.lax.Precision.HIGHEST


def _scores(q_ref, k_ref):
    q = q_ref[...]
    qn = q / jnp.maximum(jnp.sqrt(jnp.sum(q * q, axis=-1, keepdims=True)), 1e-12)
    na = jnp.maximum(jnp.sqrt(jnp.sum(qn * qn, axis=-1, keepdims=True)), 1e-8)
    k = k_ref[...]
    kn = k / jnp.maximum(jnp.sqrt(jnp.sum(k * k, axis=-1, keepdims=True)), 1e-12)
    nb = jnp.maximum(jnp.sqrt(jnp.sum(kn * kn, axis=-1, keepdims=True)), 1e-8)
    raw = lax.dot_general(qn, kn, (((1,), (1,)), ((), ())),
                          precision=_PREC, preferred_element_type=jnp.float32)
    return jnp.clip(raw / (na * nb.reshape(1, -1)), 0.0, 1.0)


def _tc_body(q1_ref, k1_ref, q2_ref, k2_ref, temp_ref, out_ref,
             att_s, m_s, z_s, thr_s):
    p = pl.program_id(0)
    t = pl.program_id(1)

    @pl.when(p == 0)
    def _phase_rowmax():
        s1 = _scores(q1_ref, k1_ref)
        s2 = _scores(q2_ref, k2_ref)

        @pl.when(t == 0)
        def _():
            m_s[...] = jnp.zeros_like(m_s)

        m_s[:, 0:1] = jnp.maximum(m_s[:, 0:1], jnp.max(s1, axis=-1, keepdims=True))
        m_s[:, 1:2] = jnp.maximum(m_s[:, 1:2], jnp.max(s2, axis=-1, keepdims=True))

    @pl.when(p == 1)
    def _phase_denom():
        s1 = _scores(q1_ref, k1_ref)
        s2 = _scores(q2_ref, k2_ref)
        e1 = jnp.exp(s1 - m_s[:, 0:1])
        e2 = jnp.exp(s2 - m_s[:, 1:2])

        @pl.when(t == 0)
        def _():
            z_s[...] = jnp.zeros_like(z_s)

        z_s[:, 0:1] = z_s[:, 0:1] + jnp.sum(e1, axis=-1, keepdims=True)
        z_s[:, 1:2] = z_s[:, 1:2] + jnp.sum(e2, axis=-1, keepdims=True)

    @pl.when(p == 2)
    def _phase_att():
        a = 1.0 / (1.0 + jnp.exp(-temp_ref[0, 0]))
        s1 = _scores(q1_ref, k1_ref)
        s2 = _scores(q2_ref, k2_ref)
        p1 = jnp.exp(s1 - m_s[:, 0:1]) / z_s[:, 0:1]
        p2 = jnp.exp(s2 - m_s[:, 1:2]) / z_s[:, 1:2]
        att_s[:, pl.ds(t * L, L)] = a * p1 + (1.0 - a) * p2

    @pl.when(p == 3)
    def _phase_out():
        @pl.when(t == 0)
        def _thresholds():
            att = att_s[...]
            rowmax = jnp.max(att, axis=-1, keepdims=True)

            def step(i, tcur):
                cand = jnp.where(att < tcur, att, -1.0)
                return jnp.max(cand, axis=-1, keepdims=True)

            t25 = lax.fori_loop(0, TOPK - 1, step, rowmax)
            thr_s[:, 0:1] = t25
            thr_s[:, 1:2] = rowmax

        att = att_s[:, pl.ds(t * L, L)]
        sel = (att >= thr_s[:, 0:1]).astype(jnp.float32)
        colmask = jnp.max(sel, axis=0, keepdims=True)
        topmask = jnp.max((att == thr_s[:, 1:2]).astype(jnp.float32),
                          axis=0, keepdims=True)
        out_ref[...] = jnp.where((colmask > 0.0) & (topmask == 0.0), att, 0.0)


@jax.jit
def kernel(q1, k1, q2, k2, temp):
    grid = (4, NT)
    return pl.pallas_call(
        _tc_body,
        grid=grid,
        in_specs=[
            pl.BlockSpec((B, D), lambda p, t: (0, 0)),
            pl.BlockSpec((L, D), lambda p, t: (t, 0)),
            pl.BlockSpec((B, D), lambda p, t: (0, 0)),
            pl.BlockSpec((L, D), lambda p, t: (t, 0)),
            pl.BlockSpec((1, 1), lambda p, t: (0, 0)),
        ],
        out_specs=pl.BlockSpec((B, L), lambda p, t: (0, t)),
        out_shape=jax.ShapeDtypeStruct((B, N), jnp.float32),
        scratch_shapes=[
            pltpu.VMEM((B, N), jnp.float32),
            pltpu.VMEM((B, 128), jnp.float32),
            pltpu.VMEM((B, 128), jnp.float32),
            pltpu.VMEM((B, 128), jnp.float32),
        ],
    )(q1, k1, q2, k2, temp.reshape(1, 1))


# speedup vs baseline: 4.4453x; 1.6048x over previous
import functools

import jax
import jax.numpy as jnp
from jax import lax
from jax.experimental import pallas as pl
from jax.experimental.pallas import tpu as pltpu
from jax.experimental.pallas import tpu_sc as plsc

B, N, D = 64, 32768, 64
TOPK = 25
L = 4096
NT = N // L
BLK = 128
NBLK = N // BLK
BPT = L // BLK
MPAD = 128
NMP = NT * MPAD
NGATH = 32

_PREC = jax.lax.Precision.HIGHEST


def _scores(q_ref, k_ref):
    q = q_ref[...]
    qn = q / jnp.maximum(jnp.sqrt(jnp.sum(q * q, axis=-1, keepdims=True)), 1e-12)
    na = jnp.maximum(jnp.sqrt(jnp.sum(qn * qn, axis=-1, keepdims=True)), 1e-8)
    k = k_ref[...]
    kn = k / jnp.maximum(jnp.sqrt(jnp.sum(k * k, axis=-1, keepdims=True)), 1e-12)
    nb = jnp.maximum(jnp.sqrt(jnp.sum(kn * kn, axis=-1, keepdims=True)), 1e-8)
    raw = lax.dot_general(qn, kn, (((1,), (1,)), ((), ())),
                          precision=_PREC, preferred_element_type=jnp.float32)
    return jnp.clip(raw / (na * nb.reshape(1, -1)), 0.0, 1.0)


def _tc_dense_body(q1_ref, k1_ref, q2_ref, k2_ref, temp_ref, att_ref, mb_ref,
                   s1_s, s2_s, m_s, z_s):
    p = pl.program_id(0)
    t = pl.program_id(1)

    @pl.when(p == 0)
    def _phase_scores():
        s1 = _scores(q1_ref, k1_ref)
        s2 = _scores(q2_ref, k2_ref)
        s1_s[:, pl.ds(t * L, L)] = s1
        s2_s[:, pl.ds(t * L, L)] = s2

        @pl.when(t == 0)
        def _():
            m_s[...] = jnp.zeros_like(m_s)

        m_s[:, 0:1] = jnp.maximum(m_s[:, 0:1], jnp.max(s1, axis=-1, keepdims=True))
        m_s[:, 1:2] = jnp.maximum(m_s[:, 1:2], jnp.max(s2, axis=-1, keepdims=True))

    @pl.when(p == 1)
    def _phase_denom():
        @pl.when(t == 0)
        def _():
            z_s[...] = jnp.zeros_like(z_s)

        e1 = jnp.exp(s1_s[:, pl.ds(t * L, L)] - m_s[:, 0:1])
        e2 = jnp.exp(s2_s[:, pl.ds(t * L, L)] - m_s[:, 1:2])
        z_s[:, 0:1] = z_s[:, 0:1] + jnp.sum(e1, axis=-1, keepdims=True)
        z_s[:, 1:2] = z_s[:, 1:2] + jnp.sum(e2, axis=-1, keepdims=True)

    @pl.when(p == 2)
    def _phase_att():
        a = 1.0 / (1.0 + jnp.exp(-temp_ref[0, 0]))
        p1 = jnp.exp(s1_s[:, pl.ds(t * L, L)] - m_s[:, 0:1]) / z_s[:, 0:1]
        p2 = jnp.exp(s2_s[:, pl.ds(t * L, L)] - m_s[:, 1:2]) / z_s[:, 1:2]
        att = a * p1 + (1.0 - a) * p2
        att_ref[...] = att
        mx = jnp.max(att.reshape(B, BPT, BLK), axis=-1)
        pad = jnp.full((B, MPAD - BPT), -1.0, jnp.float32)
        mb_ref[:, pl.ds(t * MPAD, MPAD)] = jnp.concatenate([mx, pad], axis=1)


def _tc_dense(q1, k1, q2, k2, temp):
    return pl.pallas_call(
        _tc_dense_body,
        grid=(3, NT),
        in_specs=[
            pl.BlockSpec((B, D), lambda p, t: (0, 0)),
            pl.BlockSpec((L, D), lambda p, t: (jnp.where(p == 0, t, 0), 0)),
            pl.BlockSpec((B, D), lambda p, t: (0, 0)),
            pl.BlockSpec((L, D), lambda p, t: (jnp.where(p == 0, t, 0), 0)),
            pl.BlockSpec((1, 1), lambda p, t: (0, 0)),
        ],
        out_specs=[
            pl.BlockSpec((B, L), lambda p, t: (0, jnp.where(p == 2, t, 0))),
            pl.BlockSpec((B, NMP), lambda p, t: (0, 0)),
        ],
        out_shape=[
            jax.ShapeDtypeStruct((B, N), jnp.float32),
            jax.ShapeDtypeStruct((B, NMP), jnp.float32),
        ],
        scratch_shapes=[
            pltpu.VMEM((B, N), jnp.float32),
            pltpu.VMEM((B, N), jnp.float32),
            pltpu.VMEM((B, 128), jnp.float32),
            pltpu.VMEM((B, 128), jnp.float32),
        ],
    )(q1, k1, q2, k2, temp.reshape(1, 1))


def _sc_topk_body(mb_hbm, attr_hbm, thr_hbm, mrow_v, g_v, idx_v, cand_v,
                  comp_v, out_v, sem):
    cid = lax.axis_index("c")
    sid = lax.axis_index("s")
    wid = sid * 2 + cid
    lane = lax.iota(jnp.int32, 16)
    NV = NMP // 16
    NG = NV // 16

    for r in range(2):
        b = wid * 2 + r
        pltpu.sync_copy(mb_hbm.at[b], mrow_v)

        def gbuild(j4, _):
            gvec = jnp.full((16,), -1.0, jnp.float32)
            for jj in range(16):
                mj = jnp.max(mrow_v[pl.ds(j4 * 256 + jj * 16, 16)])
                gvec = jnp.where(lane == jj, mj, gvec)
            g_v[pl.ds(j4 * 16, 16)] = gvec
            return 0

        lax.fori_loop(0, NG, gbuild, 0)

        def ext_step(i, carry):
            idx0, idx1, rowmax, t0 = carry
            g = [g_v[pl.ds(j4 * 16, 16)] for j4 in range(NG)]
            mx = g[0]
            for j4 in range(1, NG):
                mx = jnp.maximum(mx, g[j4])
            m = jnp.max(mx)
            gid = jnp.full((16,), -1, jnp.int32)
            for j4 in range(NG):
                gid = jnp.maximum(gid, jnp.where(g[j4] == m, lane + j4 * 16, -1))
            jstar = jnp.max(gid)
            v = mrow_v[pl.ds(jstar * 16, 16)]
            li = jnp.max(jnp.where(v == m, lane, -1))
            pos = jstar * 16 + li
            bid = (pos >> 7) * BPT + (pos & 127)
            v2 = jnp.where(v == m, -1.0, v)
            mrow_v[pl.ds(jstar * 16, 16)] = v2
            gnew = jnp.max(v2)
            for j4 in range(NG):
                gj = g_v[pl.ds(j4 * 16, 16)]
                g_v[pl.ds(j4 * 16, 16)] = jnp.where(
                    lane + j4 * 16 == jstar, gnew, gj)
            idx0 = jnp.where(lane == i, bid, idx0)
            idx1 = jnp.where(lane == i - 16, bid, idx1)
            rowmax = jnp.where(i == 0, m, rowmax)
            t0 = jnp.where(i == TOPK - 1, m, t0)
            return (idx0, idx1, rowmax, t0)

        zi = jnp.zeros((16,), jnp.int32)
        idx0, idx1, rowmax, t0 = lax.fori_loop(
            0, NGATH, ext_step, (zi, zi, 0.0, 0.0))

        idx_v[pl.ds(0, 16)] = idx0 + b * NBLK
        idx_v[pl.ds(16, 16)] = idx1 + b * NBLK

        pltpu.async_copy(attr_hbm.at[idx_v], cand_v, sem).wait()

        def comp_step(j, off):
            for l in range(BLK // 16):
                v = cand_v[j, pl.ds(l * 16, 16)]
                msk = v >= t0
                plsc.store_compressed(comp_v.at[pl.ds(off, 16)], v, mask=msk)
                cnt = plsc.all_reduce_population_count(msk)
                off = off + cnt[0]
            return off

        off = lax.fori_loop(0, NGATH, comp_step, jnp.int32(0))
        comp_v[pl.ds(off, 16)] = jnp.full((16,), -1.0, jnp.float32)
        nv = (off + 15) >> 4

        def chain_step(i, tprev):
            def scan_vreg(j, acc):
                v = comp_v[pl.ds(j * 16, 16)]
                return jnp.maximum(acc, jnp.where(v < tprev, v, -1.0))

            acc = lax.fori_loop(0, nv, scan_vreg,
                                jnp.full((16,), -1.0, jnp.float32))
            return jnp.max(acc)

        t25 = lax.fori_loop(0, TOPK - 1, chain_step, rowmax)

        vec = jnp.where(lane == 0, t25, jnp.where(lane == 1, rowmax, 0.0))
        out_v[r, :] = vec

    pltpu.sync_copy(out_v, thr_hbm.at[pl.ds(wid * 2, 2)])


def _sc_topk(mb, attr):
    mesh = plsc.VectorSubcoreMesh(core_axis_name="c", subcore_axis_name="s")
    f = functools.partial(
        pl.kernel,
        mesh=mesh,
        compiler_params=pltpu.CompilerParams(needs_layout_passes=False),
        out_type=jax.ShapeDtypeStruct((B, 16), jnp.float32),
        scratch_types=[
            pltpu.VMEM((NMP,), jnp.float32),
            pltpu.VMEM((NMP // 16,), jnp.float32),
            pltpu.VMEM((NGATH,), jnp.int32),
            pltpu.VMEM((NGATH, BLK), jnp.float32),
            pltpu.VMEM((NGATH * BLK + 16,), jnp.float32),
            pltpu.VMEM((2, 16), jnp.float32),
            pltpu.SemaphoreType.DMA,
        ],
    )(_sc_topk_body)
    return f(mb, attr)


def _tc_mask_body(att_ref, thr_ref, out_ref):
    att = att_ref[...]
    t25 = thr_ref[:, 0:1]
    rowmax = thr_ref[:, 1:2]
    sel = (att >= t25).astype(jnp.float32)
    colmask = jnp.max(sel, axis=0, keepdims=True)
    topmask = jnp.max((att == rowmax).astype(jnp.float32), axis=0, keepdims=True)
    out_ref[...] = jnp.where((colmask > 0.0) & (topmask == 0.0), att, 0.0)


def _tc_mask(att, thr):
    return pl.pallas_call(
        _tc_mask_body,
        grid=(NT,),
        in_specs=[
            pl.BlockSpec((B, L), lambda t: (0, t)),
            pl.BlockSpec((B, 16), lambda t: (0, 0)),
        ],
        out_specs=pl.BlockSpec((B, L), lambda t: (0, t)),
        out_shape=jax.ShapeDtypeStruct((B, N), jnp.float32),
    )(att, thr)


@jax.jit
def kernel(q1, k1, q2, k2, temp):
    att, mb = _tc_dense(q1, k1, q2, k2, temp)
    thr = _sc_topk(mb, att.reshape(B * NBLK, BLK))
    return _tc_mask(att, thr)


# speedup vs baseline: 4.8125x; 1.0826x over previous
import functools

import jax
import jax.numpy as jnp
from jax import lax
from jax.experimental import pallas as pl
from jax.experimental.pallas import tpu as pltpu
from jax.experimental.pallas import tpu_sc as plsc

B, N, D = 64, 32768, 64
TOPK = 25
L = 4096
NT = N // L
BLK = 128
NBLK = N // BLK
BPT = L // BLK
MPAD = 128
NMP = NT * MPAD
NGATH = 32

_PREC = jax.lax.Precision.HIGHEST


def _scores(q_ref, k_ref):
    q = q_ref[...]
    qn = q / jnp.maximum(jnp.sqrt(jnp.sum(q * q, axis=-1, keepdims=True)), 1e-12)
    na = jnp.maximum(jnp.sqrt(jnp.sum(qn * qn, axis=-1, keepdims=True)), 1e-8)
    k = k_ref[...]
    nrm = jnp.sqrt(jnp.sum(k * k, axis=-1, keepdims=True))
    raw = lax.dot_general(qn, k, (((1,), (1,)), ((), ())),
                          precision=_PREC, preferred_element_type=jnp.float32)
    s = raw / (na * nrm.reshape(1, -1))
    return jnp.clip(s, 0.0, 1.0)


def _tc_dense_body(q1_ref, k1_ref, q2_ref, k2_ref, temp_ref, att_ref, mb_ref,
                   s1_s, s2_s, m_s, z_s):
    p = pl.program_id(0)
    t = pl.program_id(1)

    @pl.when(p == 0)
    def _phase_scores():
        s1 = _scores(q1_ref, k1_ref)
        s2 = _scores(q2_ref, k2_ref)
        s1_s[:, pl.ds(t * L, L)] = s1
        s2_s[:, pl.ds(t * L, L)] = s2

        @pl.when(t == 0)
        def _():
            m_s[...] = jnp.zeros_like(m_s)

        m_s[:, 0:1] = jnp.maximum(m_s[:, 0:1], jnp.max(s1, axis=-1, keepdims=True))
        m_s[:, 1:2] = jnp.maximum(m_s[:, 1:2], jnp.max(s2, axis=-1, keepdims=True))

    @pl.when(p == 1)
    def _phase_denom():
        @pl.when(t == 0)
        def _():
            z_s[...] = jnp.zeros_like(z_s)

        e1 = jnp.exp(s1_s[:, pl.ds(t * L, L)] - m_s[:, 0:1])
        e2 = jnp.exp(s2_s[:, pl.ds(t * L, L)] - m_s[:, 1:2])
        z_s[:, 0:1] = z_s[:, 0:1] + jnp.sum(e1, axis=-1, keepdims=True)
        z_s[:, 1:2] = z_s[:, 1:2] + jnp.sum(e2, axis=-1, keepdims=True)

    @pl.when(p == 2)
    def _phase_att():
        a = 1.0 / (1.0 + jnp.exp(-temp_ref[0, 0]))
        p1 = jnp.exp(s1_s[:, pl.ds(t * L, L)] - m_s[:, 0:1]) / z_s[:, 0:1]
        p2 = jnp.exp(s2_s[:, pl.ds(t * L, L)] - m_s[:, 1:2]) / z_s[:, 1:2]
        att = a * p1 + (1.0 - a) * p2
        att_ref[...] = att
        mx = jnp.max(att.reshape(B, BPT, BLK), axis=-1)
        pad = jnp.full((B, MPAD - BPT), -1.0, jnp.float32)
        mb_ref[:, pl.ds(t * MPAD, MPAD)] = jnp.concatenate([mx, pad], axis=1)


def _tc_dense(q1, k1, q2, k2, temp):
    return pl.pallas_call(
        _tc_dense_body,
        grid=(3, NT),
        in_specs=[
            pl.BlockSpec((B, D), lambda p, t: (0, 0)),
            pl.BlockSpec((L, D), lambda p, t: (jnp.where(p == 0, t, 0), 0)),
            pl.BlockSpec((B, D), lambda p, t: (0, 0)),
            pl.BlockSpec((L, D), lambda p, t: (jnp.where(p == 0, t, 0), 0)),
            pl.BlockSpec((1, 1), lambda p, t: (0, 0)),
        ],
        out_specs=[
            pl.BlockSpec((B, L), lambda p, t: (0, jnp.where(p == 2, t, 0))),
            pl.BlockSpec((B, NMP), lambda p, t: (0, 0)),
        ],
        out_shape=[
            jax.ShapeDtypeStruct((B, N), jnp.float32),
            jax.ShapeDtypeStruct((B, NMP), jnp.float32),
        ],
        scratch_shapes=[
            pltpu.VMEM((B, N), jnp.float32),
            pltpu.VMEM((B, N), jnp.float32),
            pltpu.VMEM((B, 128), jnp.float32),
            pltpu.VMEM((B, 128), jnp.float32),
        ],
    )(q1, k1, q2, k2, temp.reshape(1, 1))


def _sc_topk_body(mb_hbm, attr_hbm, thr_hbm, mrow_v, g_v, idx_v, cand_v,
                  comp_v, out_v, sem):
    cid = lax.axis_index("c")
    sid = lax.axis_index("s")
    wid = sid * 2 + cid
    lane = lax.iota(jnp.int32, 16)
    NV = NMP // 16
    NG = NV // 16

    for r in range(2):
        b = wid * 2 + r
        pltpu.sync_copy(mb_hbm.at[b], mrow_v)

        def gbuild(j4, _):
            gvec = jnp.full((16,), -1.0, jnp.float32)
            for jj in range(16):
                mj = jnp.max(mrow_v[pl.ds(j4 * 256 + jj * 16, 16)])
                gvec = jnp.where(lane == jj, mj, gvec)
            g_v[pl.ds(j4 * 16, 16)] = gvec
            return 0

        lax.fori_loop(0, NG, gbuild, 0)

        def ext_step(i, carry):
            idx0, idx1, rowmax, t0 = carry
            g = [g_v[pl.ds(j4 * 16, 16)] for j4 in range(NG)]
            mx = g[0]
            for j4 in range(1, NG):
                mx = jnp.maximum(mx, g[j4])
            m = jnp.max(mx)
            gid = jnp.full((16,), -1, jnp.int32)
            for j4 in range(NG):
                gid = jnp.maximum(gid, jnp.where(g[j4] == m, lane + j4 * 16, -1))
            jstar = jnp.max(gid)
            v = mrow_v[pl.ds(jstar * 16, 16)]
            li = jnp.max(jnp.where(v == m, lane, -1))
            pos = jstar * 16 + li
            bid = (pos >> 7) * BPT + (pos & 127)
            v2 = jnp.where(v == m, -1.0, v)
            mrow_v[pl.ds(jstar * 16, 16)] = v2
            gnew = jnp.max(v2)
            for j4 in range(NG):
                gj = g_v[pl.ds(j4 * 16, 16)]
                g_v[pl.ds(j4 * 16, 16)] = jnp.where(
                    lane + j4 * 16 == jstar, gnew, gj)
            idx0 = jnp.where(lane == i, bid, idx0)
            idx1 = jnp.where(lane == i - 16, bid, idx1)
            rowmax = jnp.where(i == 0, m, rowmax)
            t0 = jnp.where(i == TOPK - 1, m, t0)
            return (idx0, idx1, rowmax, t0)

        zi = jnp.zeros((16,), jnp.int32)
        idx0, idx1, rowmax, t0 = lax.fori_loop(
            0, NGATH, ext_step, (zi, zi, 0.0, 0.0))

        idx_v[pl.ds(0, 16)] = idx0 + b * NBLK
        idx_v[pl.ds(16, 16)] = idx1 + b * NBLK

        pltpu.async_copy(attr_hbm.at[idx_v], cand_v, sem).wait()

        def comp_step(j, off):
            for l in range(BLK // 16):
                v = cand_v[j, pl.ds(l * 16, 16)]
                msk = v >= t0
                plsc.store_compressed(comp_v.at[pl.ds(off, 16)], v, mask=msk)
                cnt = plsc.all_reduce_population_count(msk)
                off = off + cnt[0]
            return off

        off = lax.fori_loop(0, NGATH, comp_step, jnp.int32(0))
        comp_v[pl.ds(off, 16)] = jnp.full((16,), -1.0, jnp.float32)
        nv = (off + 15) >> 4

        def chain_step(i, tprev):
            def scan_vreg(j, acc):
                v = comp_v[pl.ds(j * 16, 16)]
                return jnp.maximum(acc, jnp.where(v < tprev, v, -1.0))

            acc = lax.fori_loop(0, nv, scan_vreg,
                                jnp.full((16,), -1.0, jnp.float32))
            return jnp.max(acc)

        t25 = lax.fori_loop(0, TOPK - 1, chain_step, rowmax)

        vec = jnp.where(lane == 0, t25, jnp.where(lane == 1, rowmax, 0.0))
        out_v[r, :] = vec

    pltpu.sync_copy(out_v, thr_hbm.at[pl.ds(wid * 2, 2)])


def _sc_topk(mb, attr):
    mesh = plsc.VectorSubcoreMesh(core_axis_name="c", subcore_axis_name="s")
    f = functools.partial(
        pl.kernel,
        mesh=mesh,
        compiler_params=pltpu.CompilerParams(needs_layout_passes=False),
        out_type=jax.ShapeDtypeStruct((B, 16), jnp.float32),
        scratch_types=[
            pltpu.VMEM((NMP,), jnp.float32),
            pltpu.VMEM((NMP // 16,), jnp.float32),
            pltpu.VMEM((NGATH,), jnp.int32),
            pltpu.VMEM((NGATH, BLK), jnp.float32),
            pltpu.VMEM((NGATH * BLK + 16,), jnp.float32),
            pltpu.VMEM((2, 16), jnp.float32),
            pltpu.SemaphoreType.DMA,
        ],
    )(_sc_topk_body)
    return f(mb, attr)


def _tc_mask_body(att_ref, thr_ref, out_ref):
    att = att_ref[...]
    t25 = thr_ref[:, 0:1]
    rowmax = thr_ref[:, 1:2]
    sel = (att >= t25).astype(jnp.float32)
    colmask = jnp.max(sel, axis=0, keepdims=True)
    topmask = jnp.max((att == rowmax).astype(jnp.float32), axis=0, keepdims=True)
    out_ref[...] = jnp.where((colmask > 0.0) & (topmask == 0.0), att, 0.0)


def _tc_mask(att, thr):
    return pl.pallas_call(
        _tc_mask_body,
        grid=(NT,),
        in_specs=[
            pl.BlockSpec((B, L), lambda t: (0, t)),
            pl.BlockSpec((B, 16), lambda t: (0, 0)),
        ],
        out_specs=pl.BlockSpec((B, L), lambda t: (0, t)),
        out_shape=jax.ShapeDtypeStruct((B, N), jnp.float32),
    )(att, thr)


@jax.jit
def kernel(q1, k1, q2, k2, temp):
    att, mb = _tc_dense(q1, k1, q2, k2, temp)
    thr = _sc_topk(mb, att.reshape(B * NBLK, BLK))
    return _tc_mask(att, thr)


# speedup vs baseline: 7.9439x; 1.6507x over previous
import functools

import jax
import jax.numpy as jnp
from jax import lax
from jax.experimental import pallas as pl
from jax.experimental.pallas import tpu as pltpu
from jax.experimental.pallas import tpu_sc as plsc

B, N, D = 64, 32768, 64
TOPK = 25
L = 4096
NT = N // L
BLK = 128
NBLK = N // BLK
BPT = L // BLK
MPAD = 128
NMP = NT * MPAD
NGATH = 32

_PREC = jax.lax.Precision.HIGHEST


def _scores(q_ref, kt_ref):
    q = q_ref[...]
    qn = q / jnp.maximum(jnp.sqrt(jnp.sum(q * q, axis=-1, keepdims=True)), 1e-12)
    na = jnp.maximum(jnp.sqrt(jnp.sum(qn * qn, axis=-1, keepdims=True)), 1e-8)
    kt = kt_ref[...]
    nrm = jnp.sqrt(jnp.sum(kt * kt, axis=0, keepdims=True))
    raw = lax.dot_general(qn, kt, (((1,), (0,)), ((), ())),
                          precision=_PREC, preferred_element_type=jnp.float32)
    s = raw / na / nrm
    return jnp.clip(s, 0.0, 1.0)


def _tc_dense_body(q1_ref, k1_ref, q2_ref, k2_ref, temp_ref, att_ref, mb_ref,
                   s1_s, s2_s, m_s, z_s):
    p = pl.program_id(0)
    t = pl.program_id(1)

    @pl.when(p == 0)
    def _phase_scores():
        s1 = _scores(q1_ref, k1_ref)
        s2 = _scores(q2_ref, k2_ref)
        s1_s[:, pl.ds(t * L, L)] = s1
        s2_s[:, pl.ds(t * L, L)] = s2

        @pl.when(t == 0)
        def _():
            m_s[...] = jnp.zeros_like(m_s)

        m_s[:, 0:1] = jnp.maximum(m_s[:, 0:1], jnp.max(s1, axis=-1, keepdims=True))
        m_s[:, 1:2] = jnp.maximum(m_s[:, 1:2], jnp.max(s2, axis=-1, keepdims=True))

    @pl.when(p == 1)
    def _phase_denom():
        @pl.when(t == 0)
        def _():
            z_s[...] = jnp.zeros_like(z_s)

        e1 = jnp.exp(s1_s[:, pl.ds(t * L, L)] - m_s[:, 0:1])
        e2 = jnp.exp(s2_s[:, pl.ds(t * L, L)] - m_s[:, 1:2])
        z_s[:, 0:1] = z_s[:, 0:1] + jnp.sum(e1, axis=-1, keepdims=True)
        z_s[:, 1:2] = z_s[:, 1:2] + jnp.sum(e2, axis=-1, keepdims=True)

    @pl.when(p == 2)
    def _phase_att():
        a = 1.0 / (1.0 + jnp.exp(-temp_ref[0, 0]))
        p1 = jnp.exp(s1_s[:, pl.ds(t * L, L)] - m_s[:, 0:1]) / z_s[:, 0:1]
        p2 = jnp.exp(s2_s[:, pl.ds(t * L, L)] - m_s[:, 1:2]) / z_s[:, 1:2]
        att = a * p1 + (1.0 - a) * p2
        att_ref[...] = att
        mx = jnp.max(att.reshape(B, BPT, BLK), axis=-1)
        pad = jnp.full((B, MPAD - BPT), -1.0, jnp.float32)
        mb_ref[:, pl.ds(t * MPAD, MPAD)] = jnp.concatenate([mx, pad], axis=1)


def _tc_dense(q1, k1, q2, k2, temp):
    return pl.pallas_call(
        _tc_dense_body,
        grid=(3, NT),
        in_specs=[
            pl.BlockSpec((B, D), lambda p, t: (0, 0)),
            pl.BlockSpec((D, L), lambda p, t: (0, jnp.where(p == 0, t, 0))),
            pl.BlockSpec((B, D), lambda p, t: (0, 0)),
            pl.BlockSpec((D, L), lambda p, t: (0, jnp.where(p == 0, t, 0))),
            pl.BlockSpec((1, 1), lambda p, t: (0, 0)),
        ],
        out_specs=[
            pl.BlockSpec((B, L), lambda p, t: (0, jnp.where(p == 2, t, 0))),
            pl.BlockSpec((B, NMP), lambda p, t: (0, 0)),
        ],
        out_shape=[
            jax.ShapeDtypeStruct((B, N), jnp.float32),
            jax.ShapeDtypeStruct((B, NMP), jnp.float32),
        ],
        scratch_shapes=[
            pltpu.VMEM((B, N), jnp.float32),
            pltpu.VMEM((B, N), jnp.float32),
            pltpu.VMEM((B, 128), jnp.float32),
            pltpu.VMEM((B, 128), jnp.float32),
        ],
    )(q1, k1, q2, k2, temp.reshape(1, 1))


def _sc_topk_body(mb_hbm, attr_hbm, thr_hbm, mrow_v, g_v, idx_v, cand_v,
                  comp_v, out_v, sem):
    cid = lax.axis_index("c")
    sid = lax.axis_index("s")
    wid = sid * 2 + cid
    lane = lax.iota(jnp.int32, 16)
    NV = NMP // 16
    NG = NV // 16

    for r in range(2):
        b = wid * 2 + r
        pltpu.sync_copy(mb_hbm.at[b], mrow_v)

        def gbuild(j4, _):
            gvec = jnp.full((16,), -1.0, jnp.float32)
            for jj in range(16):
                mj = jnp.max(mrow_v[pl.ds(j4 * 256 + jj * 16, 16)])
                gvec = jnp.where(lane == jj, mj, gvec)
            g_v[pl.ds(j4 * 16, 16)] = gvec
            return 0

        lax.fori_loop(0, NG, gbuild, 0)

        def ext_step(i, carry):
            idx0, idx1, rowmax, t0 = carry
            g = [g_v[pl.ds(j4 * 16, 16)] for j4 in range(NG)]
            mx = g[0]
            for j4 in range(1, NG):
                mx = jnp.maximum(mx, g[j4])
            m = jnp.max(mx)
            gid = jnp.full((16,), -1, jnp.int32)
            for j4 in range(NG):
                gid = jnp.maximum(gid, jnp.where(g[j4] == m, lane + j4 * 16, -1))
            jstar = jnp.max(gid)
            v = mrow_v[pl.ds(jstar * 16, 16)]
            li = jnp.max(jnp.where(v == m, lane, -1))
            pos = jstar * 16 + li
            bid = (pos >> 7) * BPT + (pos & 127)
            v2 = jnp.where(v == m, -1.0, v)
            mrow_v[pl.ds(jstar * 16, 16)] = v2
            gnew = jnp.max(v2)
            for j4 in range(NG):
                gj = g_v[pl.ds(j4 * 16, 16)]
                g_v[pl.ds(j4 * 16, 16)] = jnp.where(
                    lane + j4 * 16 == jstar, gnew, gj)
            idx0 = jnp.where(lane == i, bid, idx0)
            idx1 = jnp.where(lane == i - 16, bid, idx1)
            rowmax = jnp.where(i == 0, m, rowmax)
            t0 = jnp.where(i == TOPK - 1, m, t0)
            return (idx0, idx1, rowmax, t0)

        zi = jnp.zeros((16,), jnp.int32)
        idx0, idx1, rowmax, t0 = lax.fori_loop(
            0, NGATH, ext_step, (zi, zi, 0.0, 0.0))

        idx_v[pl.ds(0, 16)] = idx0 + b * NBLK
        idx_v[pl.ds(16, 16)] = idx1 + b * NBLK

        pltpu.async_copy(attr_hbm.at[idx_v], cand_v, sem).wait()

        def comp_step(j, off):
            for l in range(BLK // 16):
                v = cand_v[j, pl.ds(l * 16, 16)]
                msk = v >= t0
                plsc.store_compressed(comp_v.at[pl.ds(off, 16)], v, mask=msk)
                cnt = plsc.all_reduce_population_count(msk)
                off = off + cnt[0]
            return off

        off = lax.fori_loop(0, NGATH, comp_step, jnp.int32(0))
        comp_v[pl.ds(off, 16)] = jnp.full((16,), -1.0, jnp.float32)
        nv = (off + 15) >> 4

        def chain_step(i, tprev):
            def scan_vreg(j, acc):
                v = comp_v[pl.ds(j * 16, 16)]
                return jnp.maximum(acc, jnp.where(v < tprev, v, -1.0))

            acc = lax.fori_loop(0, nv, scan_vreg,
                                jnp.full((16,), -1.0, jnp.float32))
            return jnp.max(acc)

        t25 = lax.fori_loop(0, TOPK - 1, chain_step, rowmax)

        vec = jnp.where(lane == 0, t25, jnp.where(lane == 1, rowmax, 0.0))
        out_v[r, :] = vec

    pltpu.sync_copy(out_v, thr_hbm.at[pl.ds(wid * 2, 2)])


def _sc_topk(mb, attr):
    mesh = plsc.VectorSubcoreMesh(core_axis_name="c", subcore_axis_name="s")
    f = functools.partial(
        pl.kernel,
        mesh=mesh,
        compiler_params=pltpu.CompilerParams(needs_layout_passes=False),
        out_type=jax.ShapeDtypeStruct((B, 16), jnp.float32),
        scratch_types=[
            pltpu.VMEM((NMP,), jnp.float32),
            pltpu.VMEM((NMP // 16,), jnp.float32),
            pltpu.VMEM((NGATH,), jnp.int32),
            pltpu.VMEM((NGATH, BLK), jnp.float32),
            pltpu.VMEM((NGATH * BLK + 16,), jnp.float32),
            pltpu.VMEM((2, 16), jnp.float32),
            pltpu.SemaphoreType.DMA,
        ],
    )(_sc_topk_body)
    return f(mb, attr)


def _tc_mask_body(att_ref, thr_ref, out_ref):
    att = att_ref[...]
    t25 = thr_ref[:, 0:1]
    rowmax = thr_ref[:, 1:2]
    sel = (att >= t25).astype(jnp.float32)
    colmask = jnp.max(sel, axis=0, keepdims=True)
    topmask = jnp.max((att == rowmax).astype(jnp.float32), axis=0, keepdims=True)
    out_ref[...] = jnp.where((colmask > 0.0) & (topmask == 0.0), att, 0.0)


def _tc_mask(att, thr):
    return pl.pallas_call(
        _tc_mask_body,
        grid=(NT,),
        in_specs=[
            pl.BlockSpec((B, L), lambda t: (0, t)),
            pl.BlockSpec((B, 16), lambda t: (0, 0)),
        ],
        out_specs=pl.BlockSpec((B, L), lambda t: (0, t)),
        out_shape=jax.ShapeDtypeStruct((B, N), jnp.float32),
    )(att, thr)


@jax.jit
def kernel(q1, k1, q2, k2, temp):
    att, mb = _tc_dense(q1, k1.T, q2, k2.T, temp)
    thr = _sc_topk(mb, att.reshape(B * NBLK, BLK))
    return _tc_mask(att, thr)


# speedup vs baseline: 8.0093x; 1.0082x over previous
import functools

import jax
import jax.numpy as jnp
from jax import lax
from jax.experimental import pallas as pl
from jax.experimental.pallas import tpu as pltpu
from jax.experimental.pallas import tpu_sc as plsc

B, N, D = 64, 32768, 64
TOPK = 25
L = 4096
NT = N // L
BLK = 128
NBLK = N // BLK
BPT = L // BLK
MPAD = 128
NMP = NT * MPAD
NGATH = 32

_PREC = jax.lax.Precision.HIGHEST


def _scores(q_ref, kt_ref):
    q = q_ref[...]
    qn = q / jnp.maximum(jnp.sqrt(jnp.sum(q * q, axis=-1, keepdims=True)), 1e-12)
    na = jnp.maximum(jnp.sqrt(jnp.sum(qn * qn, axis=-1, keepdims=True)), 1e-8)
    kt = kt_ref[...]
    nrm = jnp.sqrt(jnp.sum(kt * kt, axis=0, keepdims=True))
    raw = lax.dot_general(qn, kt, (((1,), (0,)), ((), ())),
                          precision=_PREC, preferred_element_type=jnp.float32)
    s = raw / na / nrm
    return jnp.clip(s, 0.0, 1.0)


def _tc_dense_body(q1_ref, k1_ref, q2_ref, k2_ref, temp_ref, att_ref, mb_ref,
                   s1_s, s2_s, m_s, z_s):
    p = pl.program_id(0)
    t = pl.program_id(1)

    @pl.when(p == 0)
    def _phase_scores():
        s1 = _scores(q1_ref, k1_ref)
        s2 = _scores(q2_ref, k2_ref)
        s1_s[:, pl.ds(t * L, L)] = s1
        s2_s[:, pl.ds(t * L, L)] = s2

        @pl.when(t == 0)
        def _():
            m_s[...] = jnp.zeros_like(m_s)

        m_s[:, 0:1] = jnp.maximum(m_s[:, 0:1], jnp.max(s1, axis=-1, keepdims=True))
        m_s[:, 1:2] = jnp.maximum(m_s[:, 1:2], jnp.max(s2, axis=-1, keepdims=True))

    @pl.when(p == 1)
    def _phase_denom():
        @pl.when(t == 0)
        def _():
            z_s[...] = jnp.zeros_like(z_s)

        e1 = jnp.exp(s1_s[:, pl.ds(t * L, L)] - m_s[:, 0:1])
        e2 = jnp.exp(s2_s[:, pl.ds(t * L, L)] - m_s[:, 1:2])
        s1_s[:, pl.ds(t * L, L)] = e1
        s2_s[:, pl.ds(t * L, L)] = e2
        z_s[:, 0:1] = z_s[:, 0:1] + jnp.sum(e1, axis=-1, keepdims=True)
        z_s[:, 1:2] = z_s[:, 1:2] + jnp.sum(e2, axis=-1, keepdims=True)

    @pl.when(p == 2)
    def _phase_att():
        a = 1.0 / (1.0 + jnp.exp(-temp_ref[0, 0]))
        p1 = s1_s[:, pl.ds(t * L, L)] / z_s[:, 0:1]
        p2 = s2_s[:, pl.ds(t * L, L)] / z_s[:, 1:2]
        att = a * p1 + (1.0 - a) * p2
        att_ref[...] = att
        mx = jnp.max(att.reshape(B, BPT, BLK), axis=-1)
        pad = jnp.full((B, MPAD - BPT), -1.0, jnp.float32)
        mb_ref[:, pl.ds(t * MPAD, MPAD)] = jnp.concatenate([mx, pad], axis=1)


def _tc_dense(q1, k1, q2, k2, temp):
    return pl.pallas_call(
        _tc_dense_body,
        grid=(3, NT),
        in_specs=[
            pl.BlockSpec((B, D), lambda p, t: (0, 0)),
            pl.BlockSpec((D, L), lambda p, t: (0, jnp.where(p == 0, t, 0))),
            pl.BlockSpec((B, D), lambda p, t: (0, 0)),
            pl.BlockSpec((D, L), lambda p, t: (0, jnp.where(p == 0, t, 0))),
            pl.BlockSpec((1, 1), lambda p, t: (0, 0)),
        ],
        out_specs=[
            pl.BlockSpec((B, L), lambda p, t: (0, jnp.where(p == 2, t, 0))),
            pl.BlockSpec((B, NMP), lambda p, t: (0, 0)),
        ],
        out_shape=[
            jax.ShapeDtypeStruct((B, N), jnp.float32),
            jax.ShapeDtypeStruct((B, NMP), jnp.float32),
        ],
        scratch_shapes=[
            pltpu.VMEM((B, N), jnp.float32),
            pltpu.VMEM((B, N), jnp.float32),
            pltpu.VMEM((B, 128), jnp.float32),
            pltpu.VMEM((B, 128), jnp.float32),
        ],
    )(q1, k1, q2, k2, temp.reshape(1, 1))


def _sc_topk_body(mb_hbm, attr_hbm, thr_hbm, mrow_v, g_v, idx_v, cand_v,
                  comp_v, out_v, sem):
    cid = lax.axis_index("c")
    sid = lax.axis_index("s")
    wid = sid * 2 + cid
    lane = lax.iota(jnp.int32, 16)
    NV = NMP // 16
    NG = NV // 16

    for r in range(2):
        b = wid * 2 + r
        pltpu.sync_copy(mb_hbm.at[b], mrow_v)

        def gbuild(j4, _):
            gvec = jnp.full((16,), -1.0, jnp.float32)
            for jj in range(16):
                mj = jnp.max(mrow_v[pl.ds(j4 * 256 + jj * 16, 16)])
                gvec = jnp.where(lane == jj, mj, gvec)
            g_v[pl.ds(j4 * 16, 16)] = gvec
            return 0

        lax.fori_loop(0, NG, gbuild, 0)

        def ext_step(i, carry):
            idx0, idx1, rowmax, t0 = carry
            g = [g_v[pl.ds(j4 * 16, 16)] for j4 in range(NG)]
            mx = g[0]
            for j4 in range(1, NG):
                mx = jnp.maximum(mx, g[j4])
            m = jnp.max(mx)
            gid = jnp.full((16,), -1, jnp.int32)
            for j4 in range(NG):
                gid = jnp.maximum(gid, jnp.where(g[j4] == m, lane + j4 * 16, -1))
            jstar = jnp.max(gid)
            v = mrow_v[pl.ds(jstar * 16, 16)]
            li = jnp.max(jnp.where(v == m, lane, -1))
            pos = jstar * 16 + li
            bid = (pos >> 7) * BPT + (pos & 127)
            v2 = jnp.where(v == m, -1.0, v)
            mrow_v[pl.ds(jstar * 16, 16)] = v2
            gnew = jnp.max(v2)
            for j4 in range(NG):
                gj = g_v[pl.ds(j4 * 16, 16)]
                g_v[pl.ds(j4 * 16, 16)] = jnp.where(
                    lane + j4 * 16 == jstar, gnew, gj)
            idx0 = jnp.where(lane == i, bid, idx0)
            idx1 = jnp.where(lane == i - 16, bid, idx1)
            rowmax = jnp.where(i == 0, m, rowmax)
            t0 = jnp.where(i == TOPK - 1, m, t0)
            return (idx0, idx1, rowmax, t0)

        zi = jnp.zeros((16,), jnp.int32)
        idx0, idx1, rowmax, t0 = lax.fori_loop(
            0, NGATH, ext_step, (zi, zi, 0.0, 0.0))

        idx_v[pl.ds(0, 16)] = idx0 + b * NBLK
        idx_v[pl.ds(16, 16)] = idx1 + b * NBLK

        pltpu.async_copy(attr_hbm.at[idx_v], cand_v, sem).wait()

        def comp_step(j, off):
            for l in range(BLK // 16):
                v = cand_v[j, pl.ds(l * 16, 16)]
                msk = v >= t0
                plsc.store_compressed(comp_v.at[pl.ds(off, 16)], v, mask=msk)
                cnt = plsc.all_reduce_population_count(msk)
                off = off + cnt[0]
            return off

        off = lax.fori_loop(0, NGATH, comp_step, jnp.int32(0))
        comp_v[pl.ds(off, 16)] = jnp.full((16,), -1.0, jnp.float32)
        nv = (off + 15) >> 4

        def chain_step(i, tprev):
            def scan_vreg(j, acc):
                v = comp_v[pl.ds(j * 16, 16)]
                return jnp.maximum(acc, jnp.where(v < tprev, v, -1.0))

            acc = lax.fori_loop(0, nv, scan_vreg,
                                jnp.full((16,), -1.0, jnp.float32))
            return jnp.max(acc)

        t25 = lax.fori_loop(0, TOPK - 1, chain_step, rowmax)

        vec = jnp.where(lane == 0, t25, jnp.where(lane == 1, rowmax, 0.0))
        out_v[r, :] = vec

    pltpu.sync_copy(out_v, thr_hbm.at[pl.ds(wid * 2, 2)])


def _sc_topk(mb, attr):
    mesh = plsc.VectorSubcoreMesh(core_axis_name="c", subcore_axis_name="s")
    f = functools.partial(
        pl.kernel,
        mesh=mesh,
        compiler_params=pltpu.CompilerParams(needs_layout_passes=False),
        out_type=jax.ShapeDtypeStruct((B, 16), jnp.float32),
        scratch_types=[
            pltpu.VMEM((NMP,), jnp.float32),
            pltpu.VMEM((NMP // 16,), jnp.float32),
            pltpu.VMEM((NGATH,), jnp.int32),
            pltpu.VMEM((NGATH, BLK), jnp.float32),
            pltpu.VMEM((NGATH * BLK + 16,), jnp.float32),
            pltpu.VMEM((2, 16), jnp.float32),
            pltpu.SemaphoreType.DMA,
        ],
    )(_sc_topk_body)
    return f(mb, attr)


def _tc_mask_body(att_ref, thr_ref, out_ref):
    att = att_ref[...]
    t25 = thr_ref[:, 0:1]
    rowmax = thr_ref[:, 1:2]
    sel = (att >= t25).astype(jnp.float32)
    colmask = jnp.max(sel, axis=0, keepdims=True)
    topmask = jnp.max((att == rowmax).astype(jnp.float32), axis=0, keepdims=True)
    out_ref[...] = jnp.where((colmask > 0.0) & (topmask == 0.0), att, 0.0)


def _tc_mask(att, thr):
    return pl.pallas_call(
        _tc_mask_body,
        grid=(NT,),
        in_specs=[
            pl.BlockSpec((B, L), lambda t: (0, t)),
            pl.BlockSpec((B, 16), lambda t: (0, 0)),
        ],
        out_specs=pl.BlockSpec((B, L), lambda t: (0, t)),
        out_shape=jax.ShapeDtypeStruct((B, N), jnp.float32),
    )(att, thr)


@jax.jit
def kernel(q1, k1, q2, k2, temp):
    att, mb = _tc_dense(q1, k1.T, q2, k2.T, temp)
    thr = _sc_topk(mb, att.reshape(B * NBLK, BLK))
    return _tc_mask(att, thr)


# speedup vs baseline: 8.7582x; 1.0935x over previous
import functools

import jax
import jax.numpy as jnp
from jax import lax
from jax.experimental import pallas as pl
from jax.experimental.pallas import tpu as pltpu
from jax.experimental.pallas import tpu_sc as plsc

B, N, D = 64, 32768, 64
TOPK = 25
L = 4096
NT = N // L
BLK = 128
NBLK = N // BLK
BPT = L // BLK
MPAD = 128
NMP = NT * MPAD
NGATH = 32

_PREC = jax.lax.Precision.HIGHEST


def _scores(q_ref, kt_ref):
    q = q_ref[...]
    qn = q / jnp.maximum(jnp.sqrt(jnp.sum(q * q, axis=-1, keepdims=True)), 1e-12)
    na = jnp.maximum(jnp.sqrt(jnp.sum(qn * qn, axis=-1, keepdims=True)), 1e-8)
    kt = kt_ref[...]
    nrm = jnp.sqrt(jnp.sum(kt * kt, axis=0, keepdims=True))
    raw = lax.dot_general(qn, kt, (((1,), (0,)), ((), ())),
                          precision=_PREC, preferred_element_type=jnp.float32)
    s = raw / na / nrm
    return jnp.clip(s, 0.0, 1.0)


def _tc_dense_body(q1_ref, k1_ref, q2_ref, k2_ref, temp_ref, att_ref, mb_ref,
                   s1_s, s2_s, z_s):
    p = pl.program_id(0)
    t = pl.program_id(1)

    @pl.when(p == 0)
    def _phase_scores():
        e1 = jnp.exp(_scores(q1_ref, k1_ref) - 1.0)
        e2 = jnp.exp(_scores(q2_ref, k2_ref) - 1.0)
        s1_s[:, pl.ds(t * L, L)] = e1
        s2_s[:, pl.ds(t * L, L)] = e2

        @pl.when(t == 0)
        def _():
            z_s[...] = jnp.zeros_like(z_s)

        z_s[:, 0:1] = z_s[:, 0:1] + jnp.sum(e1, axis=-1, keepdims=True)
        z_s[:, 1:2] = z_s[:, 1:2] + jnp.sum(e2, axis=-1, keepdims=True)

    @pl.when(p == 1)
    def _phase_att():
        a = 1.0 / (1.0 + jnp.exp(-temp_ref[0, 0]))
        p1 = s1_s[:, pl.ds(t * L, L)] / z_s[:, 0:1]
        p2 = s2_s[:, pl.ds(t * L, L)] / z_s[:, 1:2]
        att = a * p1 + (1.0 - a) * p2
        att_ref[...] = att
        mx = jnp.max(att.reshape(B, BPT, BLK), axis=-1)
        pad = jnp.full((B, MPAD - BPT), -1.0, jnp.float32)
        mb_ref[:, pl.ds(t * MPAD, MPAD)] = jnp.concatenate([mx, pad], axis=1)


def _tc_dense(q1, k1, q2, k2, temp):
    return pl.pallas_call(
        _tc_dense_body,
        grid=(2, NT),
        in_specs=[
            pl.BlockSpec((B, D), lambda p, t: (0, 0)),
            pl.BlockSpec((D, L), lambda p, t: (0, jnp.where(p == 0, t, 0))),
            pl.BlockSpec((B, D), lambda p, t: (0, 0)),
            pl.BlockSpec((D, L), lambda p, t: (0, jnp.where(p == 0, t, 0))),
            pl.BlockSpec((1, 1), lambda p, t: (0, 0)),
        ],
        out_specs=[
            pl.BlockSpec((B, L), lambda p, t: (0, jnp.where(p == 1, t, 0))),
            pl.BlockSpec((B, NMP), lambda p, t: (0, 0)),
        ],
        out_shape=[
            jax.ShapeDtypeStruct((B, N), jnp.float32),
            jax.ShapeDtypeStruct((B, NMP), jnp.float32),
        ],
        scratch_shapes=[
            pltpu.VMEM((B, N), jnp.float32),
            pltpu.VMEM((B, N), jnp.float32),
            pltpu.VMEM((B, 128), jnp.float32),
        ],
    )(q1, k1, q2, k2, temp.reshape(1, 1))


def _sc_topk_body(mb_hbm, attr_hbm, thr_hbm, mrow_v, g_v, idx_v, cand_v,
                  comp_v, out_v, sem):
    cid = lax.axis_index("c")
    sid = lax.axis_index("s")
    wid = sid * 2 + cid
    lane = lax.iota(jnp.int32, 16)
    NV = NMP // 16
    NG = NV // 16

    for r in range(2):
        b = wid * 2 + r
        pltpu.sync_copy(mb_hbm.at[b], mrow_v)

        def gbuild(j4, _):
            gvec = jnp.full((16,), -1.0, jnp.float32)
            for jj in range(16):
                mj = jnp.max(mrow_v[pl.ds(j4 * 256 + jj * 16, 16)])
                gvec = jnp.where(lane == jj, mj, gvec)
            g_v[pl.ds(j4 * 16, 16)] = gvec
            return 0

        lax.fori_loop(0, NG, gbuild, 0)

        def ext_step(i, carry):
            idx0, idx1, rowmax, t0 = carry
            g = [g_v[pl.ds(j4 * 16, 16)] for j4 in range(NG)]
            mx = g[0]
            for j4 in range(1, NG):
                mx = jnp.maximum(mx, g[j4])
            m = jnp.max(mx)
            gid = jnp.full((16,), -1, jnp.int32)
            for j4 in range(NG):
                gid = jnp.maximum(gid, jnp.where(g[j4] == m, lane + j4 * 16, -1))
            jstar = jnp.max(gid)
            v = mrow_v[pl.ds(jstar * 16, 16)]
            li = jnp.max(jnp.where(v == m, lane, -1))
            pos = jstar * 16 + li
            bid = (pos >> 7) * BPT + (pos & 127)
            v2 = jnp.where(v == m, -1.0, v)
            mrow_v[pl.ds(jstar * 16, 16)] = v2
            gnew = jnp.max(v2)
            for j4 in range(NG):
                gj = g_v[pl.ds(j4 * 16, 16)]
                g_v[pl.ds(j4 * 16, 16)] = jnp.where(
                    lane + j4 * 16 == jstar, gnew, gj)
            idx0 = jnp.where(lane == i, bid, idx0)
            idx1 = jnp.where(lane == i - 16, bid, idx1)
            rowmax = jnp.where(i == 0, m, rowmax)
            t0 = jnp.where(i == TOPK - 1, m, t0)
            return (idx0, idx1, rowmax, t0)

        zi = jnp.zeros((16,), jnp.int32)
        idx0, idx1, rowmax, t0 = lax.fori_loop(
            0, NGATH, ext_step, (zi, zi, 0.0, 0.0))

        idx_v[pl.ds(0, 16)] = idx0 + b * NBLK
        idx_v[pl.ds(16, 16)] = idx1 + b * NBLK

        pltpu.async_copy(attr_hbm.at[idx_v], cand_v, sem).wait()

        def comp_step(j, off):
            for l in range(BLK // 16):
                v = cand_v[j, pl.ds(l * 16, 16)]
                msk = v >= t0
                plsc.store_compressed(comp_v.at[pl.ds(off, 16)], v, mask=msk)
                cnt = plsc.all_reduce_population_count(msk)
                off = off + cnt[0]
            return off

        off = lax.fori_loop(0, TOPK, comp_step, jnp.int32(0))
        comp_v[pl.ds(off, 16)] = jnp.full((16,), -1.0, jnp.float32)
        nv = (off + 15) >> 4

        def chain_step(i, tprev):
            def scan_vreg(j, acc):
                v = comp_v[pl.ds(j * 16, 16)]
                return jnp.maximum(acc, jnp.where(v < tprev, v, -1.0))

            acc = lax.fori_loop(0, nv, scan_vreg,
                                jnp.full((16,), -1.0, jnp.float32))
            return jnp.max(acc)

        t25 = lax.fori_loop(0, TOPK - 1, chain_step, rowmax)

        vec = jnp.where(lane == 0, t25, jnp.where(lane == 1, rowmax, 0.0))
        out_v[r, :] = vec

    pltpu.sync_copy(out_v, thr_hbm.at[pl.ds(wid * 2, 2)])


def _sc_topk(mb, attr):
    mesh = plsc.VectorSubcoreMesh(core_axis_name="c", subcore_axis_name="s")
    f = functools.partial(
        pl.kernel,
        mesh=mesh,
        compiler_params=pltpu.CompilerParams(needs_layout_passes=False),
        out_type=jax.ShapeDtypeStruct((B, 16), jnp.float32),
        scratch_types=[
            pltpu.VMEM((NMP,), jnp.float32),
            pltpu.VMEM((NMP // 16,), jnp.float32),
            pltpu.VMEM((NGATH,), jnp.int32),
            pltpu.VMEM((NGATH, BLK), jnp.float32),
            pltpu.VMEM((NGATH * BLK + 16,), jnp.float32),
            pltpu.VMEM((2, 16), jnp.float32),
            pltpu.SemaphoreType.DMA,
        ],
    )(_sc_topk_body)
    return f(mb, attr)


def _tc_mask_body(att_ref, thr_ref, out_ref):
    att = att_ref[...]
    t25 = thr_ref[:, 0:1]
    rowmax = thr_ref[:, 1:2]
    sel = (att >= t25).astype(jnp.float32)
    colmask = jnp.max(sel, axis=0, keepdims=True)
    topmask = jnp.max((att == rowmax).astype(jnp.float32), axis=0, keepdims=True)
    out_ref[...] = jnp.where((colmask > 0.0) & (topmask == 0.0), att, 0.0)


L2 = 8192


def _tc_mask(att, thr):
    return pl.pallas_call(
        _tc_mask_body,
        grid=(N // L2,),
        in_specs=[
            pl.BlockSpec((B, L2), lambda t: (0, t)),
            pl.BlockSpec((B, 16), lambda t: (0, 0)),
        ],
        out_specs=pl.BlockSpec((B, L2), lambda t: (0, t)),
        out_shape=jax.ShapeDtypeStruct((B, N), jnp.float32),
    )(att, thr)


@jax.jit
def kernel(q1, k1, q2, k2, temp):
    att, mb = _tc_dense(q1, k1.T, q2, k2.T, temp)
    thr = _sc_topk(mb, att.reshape(B * NBLK, BLK))
    return _tc_mask(att, thr)
